# baseline (speedup 1.0000x reference)
import functools

import jax
import jax.numpy as jnp
from jax import lax
from jax.experimental import pallas as pl
from jax.experimental.pallas import tpu as pltpu
from jax.experimental.pallas import tpu_sc as plsc

T = 2048
H = 1024
E = 8
K = 2
FF = 4096
LBW = 0.01

M = 512
LOG2M = 9
NT = 15
NTM = NT * M
FC = 1024
NF = FF // FC

NC = 2
NS = 16
NW = NC * NS
TPW = T // NW
CH = 32
NCH = TPW // CH


def _router_body(x_ref, wr_ref, br_ref, pos_ref, w0b_ref, w1b_ref, texp_ref,
                 loss_ref):
    xb = x_ref[...]
    logits = jnp.dot(xb, wr_ref[...].T,
                     preferred_element_type=jnp.float32) + br_ref[...]
    m = jnp.max(logits, axis=-1, keepdims=True)
    p = jnp.exp(logits - m)
    p = p / jnp.sum(p, axis=-1, keepdims=True)

    iota = lax.broadcasted_iota(jnp.int32, p.shape, 1)
    m1 = jnp.max(p, axis=-1, keepdims=True)
    i1 = jnp.min(jnp.where(p == m1, iota, E), axis=-1, keepdims=True)
    pm = jnp.where(iota == i1, -1.0, p)
    m2 = jnp.max(pm, axis=-1, keepdims=True)
    i2 = jnp.min(jnp.where(pm == m2, iota, E), axis=-1, keepdims=True)
    denom = m1 + m2 + 1e-8
    mask1 = (iota == i1).astype(jnp.float32)
    mask2 = (iota == i2).astype(jnp.float32)
    w0b_ref[...] = jnp.broadcast_to(m1 / denom, (T, 16))
    w1b_ref[...] = jnp.broadcast_to(m2 / denom, (T, 16))

    s = mask1 + mask2
    ci = jnp.sum(s, axis=0, keepdims=True).astype(jnp.int32)
    pci = ((ci + (M - 1)) >> LOG2M) << LOG2M
    pcf = pci.astype(jnp.float32)
    er = lax.broadcasted_iota(jnp.int32, (E, E), 0)
    ec = lax.broadcasted_iota(jnp.int32, (E, E), 1)
    lt8 = (er < ec).astype(jnp.float32)
    off = jnp.dot(pcf, lt8, preferred_element_type=jnp.float32)

    BL = 128
    rr = lax.broadcasted_iota(jnp.int32, (BL, BL), 0)
    cc = lax.broadcasted_iota(jnp.int32, (BL, BL), 1)
    ltb = (cc < rr).astype(jnp.float32)
    run = jnp.zeros((1, E), jnp.float32)
    ranks = []
    for b in range(T // BL):
        sb = s[b * BL:(b + 1) * BL, :]
        ranks.append(jnp.dot(ltb, sb, preferred_element_type=jnp.float32) + run)
        run = run + jnp.sum(sb, axis=0, keepdims=True)
    rank = jnp.concatenate(ranks, axis=0)

    base = off + rank
    pos1 = jnp.sum(mask1 * base, axis=-1, keepdims=True)
    pos2 = jnp.sum(mask2 * (base + mask1), axis=-1, keepdims=True)
    pos_ref[...] = jnp.concatenate([pos1, pos2], axis=1).astype(jnp.int32)

    jm = lax.broadcasted_iota(jnp.int32, (1, NT), 1).astype(jnp.float32) * M
    ei = lax.broadcasted_iota(jnp.int32, (1, E), 1)
    texp = jnp.zeros((1, NT), jnp.float32)
    for e in range(E):
        sel = (ei == e).astype(jnp.float32)
        off_e = jnp.sum(off * sel, axis=-1, keepdims=True)
        pc_e = jnp.sum(pcf * sel, axis=-1, keepdims=True)
        texp += jnp.where((jm >= off_e) & (jm < off_e + pc_e), float(e), 0.0)
    total = jnp.sum(pcf, axis=-1, keepdims=True)
    last_e = jnp.max(jnp.where(pci > 0, ei, 0), axis=-1, keepdims=True)
    texp_ref[...] = jnp.where(jm < total, texp,
                              last_e.astype(jnp.float32)).astype(jnp.int32)

    avg = jnp.sum(p, axis=0, keepdims=True) / T
    mean = jnp.sum(avg) / E
    var = jnp.sum((avg - mean) ** 2) / (E - 1)
    loss_ref[...] = jnp.broadcast_to(LBW * var, (1, 1))


def _router(x2d, Wr, br2):
    return pl.pallas_call(
        _router_body,
        in_specs=[
            pl.BlockSpec((T, H), lambda: (0, 0)),
            pl.BlockSpec((E, H), lambda: (0, 0)),
            pl.BlockSpec((1, E), lambda: (0, 0)),
        ],
        out_specs=[
            pl.BlockSpec((T, K), lambda: (0, 0)),
            pl.BlockSpec((T, 16), lambda: (0, 0)),
            pl.BlockSpec((T, 16), lambda: (0, 0)),
            pl.BlockSpec((1, NT), lambda: (0, 0)),
            pl.BlockSpec((1, 1), lambda: (0, 0)),
        ],
        out_shape=[
            jax.ShapeDtypeStruct((T, K), jnp.int32),
            jax.ShapeDtypeStruct((T, 16), jnp.float32),
            jax.ShapeDtypeStruct((T, 16), jnp.float32),
            jax.ShapeDtypeStruct((1, NT), jnp.int32),
            jax.ShapeDtypeStruct((1, 1), jnp.float32),
        ],
    )(x2d, Wr, br2)


def _dispatch_body(x_hbm, pos_hbm, xg_hbm, idx0_v, idx1_v, rows_v, s0, s1):
    wid = lax.axis_index("s") * NC + lax.axis_index("c")
    tbase = wid * TPW
    pltpu.sync_copy(pos_hbm.at[0, pl.ds(tbase, TPW)], idx0_v)
    pltpu.sync_copy(pos_hbm.at[1, pl.ds(tbase, TPW)], idx1_v)
    pltpu.sync_copy(x_hbm.at[pl.ds(tbase, TPW), :], rows_v)
    c0 = pltpu.async_copy(rows_v, xg_hbm.at[idx0_v], s0)
    c1 = pltpu.async_copy(rows_v, xg_hbm.at[idx1_v], s1)
    c0.wait()
    c1.wait()


@functools.cache
def _make_dispatch():
    return pl.kernel(
        _dispatch_body,
        mesh=plsc.VectorSubcoreMesh(core_axis_name="c", subcore_axis_name="s"),
        out_type=jax.ShapeDtypeStruct((NTM, H), jnp.float32),
        scratch_types=[
            pltpu.VMEM((TPW,), jnp.int32),
            pltpu.VMEM((TPW,), jnp.int32),
            pltpu.VMEM((TPW, H), jnp.float32),
            pltpu.SemaphoreType.DMA,
            pltpu.SemaphoreType.DMA,
        ],
    )


def _gmm_body(texp_ref, xg_ref, w1_ref, b1_ref, w2_ref, b2_ref, out_ref):
    f = pl.program_id(1)
    h = jnp.dot(xg_ref[...], w1_ref[0].T, preferred_element_type=jnp.float32)
    h = h + b1_ref[0, 0]
    h = 0.5 * h * (1.0 + lax.erf(h * 0.7071067811865476))
    w2c = w2_ref[0, :, pl.ds(f * FC, FC)]
    yc = jnp.dot(h, w2c.T, preferred_element_type=jnp.float32)

    @pl.when(f == 0)
    def _():
        out_ref[...] = yc + b2_ref[0]

    @pl.when(f != 0)
    def _():
        out_ref[...] += yc


def _gmm(texp, x_g, W1, b1, W2, b2):
    grid_spec = pltpu.PrefetchScalarGridSpec(
        num_scalar_prefetch=1,
        grid=(NT, NF),
        in_specs=[
            pl.BlockSpec((M, H), lambda j, f, tx: (j, 0)),
            pl.BlockSpec((1, FC, H), lambda j, f, tx: (tx[j], f, 0)),
            pl.BlockSpec((1, 1, 1, FC), lambda j, f, tx: (tx[j], f, 0, 0)),
            pl.BlockSpec((1, H, FF), lambda j, f, tx: (tx[j], 0, 0)),
            pl.BlockSpec((1, 1, H), lambda j, f, tx: (tx[j], 0, 0)),
        ],
        out_specs=pl.BlockSpec((M, H), lambda j, f, tx: (j, 0)),
    )
    return pl.pallas_call(
        _gmm_body,
        grid_spec=grid_spec,
        out_shape=jax.ShapeDtypeStruct((NTM, H), jnp.float32),
    )(texp, x_g, W1, b1.reshape(E, NF, 1, FC), W2, b2.reshape(E, 1, H))


def _combine_body(yg_hbm, pos_hbm, w0b_hbm, w1b_hbm, out_hbm,
                  idx0_v, idx1_v, w0_v, w1_v, r0_v, r1_v, ob_v, s0, s1):
    wid = lax.axis_index("s") * NC + lax.axis_index("c")
    tbase = wid * TPW
    for half in range(NCH):
        base = tbase + half * CH
        pltpu.sync_copy(pos_hbm.at[0, pl.ds(base, CH)], idx0_v)
        pltpu.sync_copy(pos_hbm.at[1, pl.ds(base, CH)], idx1_v)
        pltpu.sync_copy(w0b_hbm.at[pl.ds(base, CH), :], w0_v)
        pltpu.sync_copy(w1b_hbm.at[pl.ds(base, CH), :], w1_v)
        c0 = pltpu.async_copy(yg_hbm.at[idx0_v], r0_v, s0)
        c1 = pltpu.async_copy(yg_hbm.at[idx1_v], r1_v, s1)
        c0.wait()
        c1.wait()
        for tok in range(CH):
            w0s = w0_v[tok, :]
            w1s = w1_v[tok, :]

            def body(j, _, tok=tok, w0s=w0s, w1s=w1s):
                a = r0_v[tok, pl.ds(j * 16, 16)]
                b = r1_v[tok, pl.ds(j * 16, 16)]
                ob_v[tok, pl.ds(j * 16, 16)] = a * w0s + b * w1s
                return 0

            lax.fori_loop(0, H // 16, body, 0)
        pltpu.sync_copy(ob_v, out_hbm.at[pl.ds(base, CH), :])


@functools.cache
def _make_combine():
    return pl.kernel(
        _combine_body,
        mesh=plsc.VectorSubcoreMesh(core_axis_name="c", subcore_axis_name="s"),
        out_type=jax.ShapeDtypeStruct((T, H), jnp.float32),
        scratch_types=[
            pltpu.VMEM((CH,), jnp.int32),
            pltpu.VMEM((CH,), jnp.int32),
            pltpu.VMEM((CH, 16), jnp.float32),
            pltpu.VMEM((CH, 16), jnp.float32),
            pltpu.VMEM((CH, H), jnp.float32),
            pltpu.VMEM((CH, H), jnp.float32),
            pltpu.VMEM((CH, H), jnp.float32),
            pltpu.SemaphoreType.DMA,
            pltpu.SemaphoreType.DMA,
        ],
    )


@jax.jit
def _moe(x2d, Wr, br2, W1, b1, W2, b2):
    pos, w0b, w1b, texp, loss = _router(x2d, Wr, br2)
    pos2t = pos.T
    x_g = _make_dispatch()(x2d, pos2t)
    y_g = _gmm(texp.reshape(NT), x_g, W1, b1, W2, b2)
    out = _make_combine()(y_g, pos2t, w0b, w1b)
    return out, loss


def kernel(x, Wr, br, W1, b1, W2, b2):
    Bq, Sq, Hq = x.shape
    x2d = x.reshape(-1, Hq)
    br2 = br.reshape(1, E)
    out, loss = _moe(x2d, Wr, br2, W1, b1, W2, b2)
    return out.reshape(Bq, Sq, Hq), loss.reshape(())


# speedup vs baseline: 1.0488x; 1.0488x over previous
import functools

import jax
import jax.numpy as jnp
from jax import lax
from jax.experimental import pallas as pl
from jax.experimental.pallas import tpu as pltpu
from jax.experimental.pallas import tpu_sc as plsc

T = 2048
H = 1024
E = 8
K = 2
FF = 4096
LBW = 0.01

M = 512
LOG2M = 9
NT = 15
NTM = NT * M
FC = 1024
NF = FF // FC

NC = 2
NS = 16
NW = NC * NS
TPW = T // NW
CH = 32
NCH = TPW // CH


def _router_body(x_ref, wr_ref, br_ref, pos_ref, w0b_ref, w1b_ref, texp_ref,
                 loss_ref):
    xb = x_ref[...]
    logits = jnp.dot(xb, wr_ref[...].T,
                     preferred_element_type=jnp.float32) + br_ref[...]
    m = jnp.max(logits, axis=-1, keepdims=True)
    p = jnp.exp(logits - m)
    p = p / jnp.sum(p, axis=-1, keepdims=True)

    iota = lax.broadcasted_iota(jnp.int32, p.shape, 1)
    m1 = jnp.max(p, axis=-1, keepdims=True)
    i1 = jnp.min(jnp.where(p == m1, iota, E), axis=-1, keepdims=True)
    pm = jnp.where(iota == i1, -1.0, p)
    m2 = jnp.max(pm, axis=-1, keepdims=True)
    i2 = jnp.min(jnp.where(pm == m2, iota, E), axis=-1, keepdims=True)
    denom = m1 + m2 + 1e-8
    mask1 = (iota == i1).astype(jnp.float32)
    mask2 = (iota == i2).astype(jnp.float32)
    w0b_ref[...] = jnp.broadcast_to(m1 / denom, (T, 16))
    w1b_ref[...] = jnp.broadcast_to(m2 / denom, (T, 16))

    s = mask1 + mask2
    ci = jnp.sum(s, axis=0, keepdims=True).astype(jnp.int32)
    pci = ((ci + (M - 1)) >> LOG2M) << LOG2M
    pcf = pci.astype(jnp.float32)
    er = lax.broadcasted_iota(jnp.int32, (E, E), 0)
    ec = lax.broadcasted_iota(jnp.int32, (E, E), 1)
    lt8 = (er < ec).astype(jnp.float32)
    off = jnp.dot(pcf, lt8, preferred_element_type=jnp.float32)

    BL = 128
    rr = lax.broadcasted_iota(jnp.int32, (BL, BL), 0)
    cc = lax.broadcasted_iota(jnp.int32, (BL, BL), 1)
    ltb = (cc < rr).astype(jnp.float32)
    run = jnp.zeros((1, E), jnp.float32)
    ranks = []
    for b in range(T // BL):
        sb = s[b * BL:(b + 1) * BL, :]
        ranks.append(jnp.dot(ltb, sb, preferred_element_type=jnp.float32) + run)
        run = run + jnp.sum(sb, axis=0, keepdims=True)
    rank = jnp.concatenate(ranks, axis=0)

    base = off + rank
    pos1 = jnp.sum(mask1 * base, axis=-1, keepdims=True)
    pos2 = jnp.sum(mask2 * (base + mask1), axis=-1, keepdims=True)
    pos_ref[...] = jnp.concatenate([pos1, pos2], axis=1).astype(jnp.int32)

    jm = lax.broadcasted_iota(jnp.int32, (1, NT), 1).astype(jnp.float32) * M
    ei = lax.broadcasted_iota(jnp.int32, (1, E), 1)
    texp = jnp.zeros((1, NT), jnp.float32)
    for e in range(E):
        sel = (ei == e).astype(jnp.float32)
        off_e = jnp.sum(off * sel, axis=-1, keepdims=True)
        pc_e = jnp.sum(pcf * sel, axis=-1, keepdims=True)
        texp += jnp.where((jm >= off_e) & (jm < off_e + pc_e), float(e), 0.0)
    total = jnp.sum(pcf, axis=-1, keepdims=True)
    last_e = jnp.max(jnp.where(pci > 0, ei, 0), axis=-1, keepdims=True)
    texp_ref[...] = jnp.where(jm < total, texp,
                              last_e.astype(jnp.float32)).astype(jnp.int32)

    avg = jnp.sum(p, axis=0, keepdims=True) / T
    mean = jnp.sum(avg) / E
    var = jnp.sum((avg - mean) ** 2) / (E - 1)
    loss_ref[...] = jnp.broadcast_to(LBW * var, (1, 1))


def _router(x2d, Wr, br2):
    return pl.pallas_call(
        _router_body,
        in_specs=[
            pl.BlockSpec((T, H), lambda: (0, 0)),
            pl.BlockSpec((E, H), lambda: (0, 0)),
            pl.BlockSpec((1, E), lambda: (0, 0)),
        ],
        out_specs=[
            pl.BlockSpec((T, K), lambda: (0, 0)),
            pl.BlockSpec((T, 16), lambda: (0, 0)),
            pl.BlockSpec((T, 16), lambda: (0, 0)),
            pl.BlockSpec((1, NT), lambda: (0, 0)),
            pl.BlockSpec((1, 1), lambda: (0, 0)),
        ],
        out_shape=[
            jax.ShapeDtypeStruct((T, K), jnp.int32),
            jax.ShapeDtypeStruct((T, 16), jnp.float32),
            jax.ShapeDtypeStruct((T, 16), jnp.float32),
            jax.ShapeDtypeStruct((1, NT), jnp.int32),
            jax.ShapeDtypeStruct((1, 1), jnp.float32),
        ],
    )(x2d, Wr, br2)


def _dispatch_body(x_hbm, pos_hbm, xg_hbm, idx0_v, idx1_v, rows_v, s0, s1):
    wid = lax.axis_index("s") * NC + lax.axis_index("c")
    tbase = wid * TPW
    pltpu.sync_copy(pos_hbm.at[0, pl.ds(tbase, TPW)], idx0_v)
    pltpu.sync_copy(pos_hbm.at[1, pl.ds(tbase, TPW)], idx1_v)
    pltpu.sync_copy(x_hbm.at[pl.ds(tbase, TPW), :], rows_v)
    c0 = pltpu.async_copy(rows_v, xg_hbm.at[idx0_v], s0)
    c1 = pltpu.async_copy(rows_v, xg_hbm.at[idx1_v], s1)
    c0.wait()
    c1.wait()


@functools.cache
def _make_dispatch():
    return pl.kernel(
        _dispatch_body,
        mesh=plsc.VectorSubcoreMesh(core_axis_name="c", subcore_axis_name="s"),
        out_type=jax.ShapeDtypeStruct((NTM, H), jnp.float32),
        scratch_types=[
            pltpu.VMEM((TPW,), jnp.int32),
            pltpu.VMEM((TPW,), jnp.int32),
            pltpu.VMEM((TPW, H), jnp.float32),
            pltpu.SemaphoreType.DMA,
            pltpu.SemaphoreType.DMA,
        ],
    )


def _gmm_body(texp_ref, xg_ref, w1_ref, b1_ref, w2_ref, b2_ref, out_ref):
    f = pl.program_id(1)
    h = jnp.dot(xg_ref[...], w1_ref[0].T, preferred_element_type=jnp.float32)
    h = h + b1_ref[0, 0]
    h = 0.5 * h * (1.0 + lax.erf(h * 0.7071067811865476))
    yc = jnp.dot(h, w2_ref[0].T, preferred_element_type=jnp.float32)

    @pl.when(f == 0)
    def _():
        out_ref[...] = yc + b2_ref[0]

    @pl.when(f != 0)
    def _():
        out_ref[...] += yc


def _gmm(texp, x_g, W1, b1, W2, b2):
    grid_spec = pltpu.PrefetchScalarGridSpec(
        num_scalar_prefetch=1,
        grid=(NT, NF),
        in_specs=[
            pl.BlockSpec((M, H), lambda j, f, tx: (j, 0)),
            pl.BlockSpec((1, FC, H), lambda j, f, tx: (tx[j], f, 0)),
            pl.BlockSpec((1, 1, 1, FC), lambda j, f, tx: (tx[j], f, 0, 0)),
            pl.BlockSpec((1, H, FC), lambda j, f, tx: (tx[j], 0, f)),
            pl.BlockSpec((1, 1, H), lambda j, f, tx: (tx[j], 0, 0)),
        ],
        out_specs=pl.BlockSpec((M, H), lambda j, f, tx: (j, 0)),
    )
    return pl.pallas_call(
        _gmm_body,
        grid_spec=grid_spec,
        out_shape=jax.ShapeDtypeStruct((NTM, H), jnp.float32),
    )(texp, x_g, W1, b1.reshape(E, NF, 1, FC), W2, b2.reshape(E, 1, H))


def _combine_body(yg_hbm, pos_hbm, w0b_hbm, w1b_hbm, out_hbm,
                  idx0_v, idx1_v, w0_v, w1_v, r0_v, r1_v, ob_v, s0, s1):
    wid = lax.axis_index("s") * NC + lax.axis_index("c")
    tbase = wid * TPW
    for half in range(NCH):
        base = tbase + half * CH
        pltpu.sync_copy(pos_hbm.at[0, pl.ds(base, CH)], idx0_v)
        pltpu.sync_copy(pos_hbm.at[1, pl.ds(base, CH)], idx1_v)
        pltpu.sync_copy(w0b_hbm.at[pl.ds(base, CH), :], w0_v)
        pltpu.sync_copy(w1b_hbm.at[pl.ds(base, CH), :], w1_v)
        c0 = pltpu.async_copy(yg_hbm.at[idx0_v], r0_v, s0)
        c1 = pltpu.async_copy(yg_hbm.at[idx1_v], r1_v, s1)
        c0.wait()
        c1.wait()
        for tok in range(CH):
            w0s = w0_v[tok, :]
            w1s = w1_v[tok, :]

            def body(j, _, tok=tok, w0s=w0s, w1s=w1s):
                for u in range(4):
                    o = j * 64 + u * 16
                    a = r0_v[tok, pl.ds(o, 16)]
                    b = r1_v[tok, pl.ds(o, 16)]
                    ob_v[tok, pl.ds(o, 16)] = a * w0s + b * w1s
                return 0

            lax.fori_loop(0, H // 64, body, 0)
        pltpu.sync_copy(ob_v, out_hbm.at[pl.ds(base, CH), :])


@functools.cache
def _make_combine():
    return pl.kernel(
        _combine_body,
        mesh=plsc.VectorSubcoreMesh(core_axis_name="c", subcore_axis_name="s"),
        out_type=jax.ShapeDtypeStruct((T, H), jnp.float32),
        scratch_types=[
            pltpu.VMEM((CH,), jnp.int32),
            pltpu.VMEM((CH,), jnp.int32),
            pltpu.VMEM((CH, 16), jnp.float32),
            pltpu.VMEM((CH, 16), jnp.float32),
            pltpu.VMEM((CH, H), jnp.float32),
            pltpu.VMEM((CH, H), jnp.float32),
            pltpu.VMEM((CH, H), jnp.float32),
            pltpu.SemaphoreType.DMA,
            pltpu.SemaphoreType.DMA,
        ],
    )


@jax.jit
def _moe(x2d, Wr, br2, W1, b1, W2, b2):
    pos, w0b, w1b, texp, loss = _router(x2d, Wr, br2)
    pos2t = pos.T
    x_g = _make_dispatch()(x2d, pos2t)
    y_g = _gmm(texp.reshape(NT), x_g, W1, b1, W2, b2)
    out = _make_combine()(y_g, pos2t, w0b, w1b)
    return out, loss


def kernel(x, Wr, br, W1, b1, W2, b2):
    Bq, Sq, Hq = x.shape
    x2d = x.reshape(-1, Hq)
    br2 = br.reshape(1, E)
    out, loss = _moe(x2d, Wr, br2, W1, b1, W2, b2)
    return out.reshape(Bq, Sq, Hq), loss.reshape(())


# speedup vs baseline: 1.0573x; 1.0080x over previous
import functools

import jax
import jax.numpy as jnp
from jax import lax
from jax.experimental import pallas as pl
from jax.experimental.pallas import tpu as pltpu
from jax.experimental.pallas import tpu_sc as plsc

T = 2048
H = 1024
E = 8
K = 2
FF = 4096
LBW = 0.01

M = 512
LOG2M = 9
NT = 15
NTM = NT * M
FC = 1024
NF = FF // FC

NC = 2
NS = 16
NW = NC * NS
TPW = T // NW
CH = 32
NCH = TPW // CH


def _router_body(x_ref, wr_ref, br_ref, pos_ref, w0b_ref, w1b_ref, texp_ref,
                 loss_ref):
    xb = x_ref[...]
    logits = jnp.dot(xb, wr_ref[...].T,
                     preferred_element_type=jnp.float32) + br_ref[...]
    m = jnp.max(logits, axis=-1, keepdims=True)
    p = jnp.exp(logits - m)
    p = p / jnp.sum(p, axis=-1, keepdims=True)

    iota = lax.broadcasted_iota(jnp.int32, p.shape, 1)
    m1 = jnp.max(p, axis=-1, keepdims=True)
    i1 = jnp.min(jnp.where(p == m1, iota, E), axis=-1, keepdims=True)
    pm = jnp.where(iota == i1, -1.0, p)
    m2 = jnp.max(pm, axis=-1, keepdims=True)
    i2 = jnp.min(jnp.where(pm == m2, iota, E), axis=-1, keepdims=True)
    denom = m1 + m2 + 1e-8
    mask1 = (iota == i1).astype(jnp.float32)
    mask2 = (iota == i2).astype(jnp.float32)
    w0b_ref[...] = jnp.broadcast_to(m1 / denom, (T, 16))
    w1b_ref[...] = jnp.broadcast_to(m2 / denom, (T, 16))

    s = mask1 + mask2
    ci = jnp.sum(s, axis=0, keepdims=True).astype(jnp.int32)
    pci = ((ci + (M - 1)) >> LOG2M) << LOG2M
    pcf = pci.astype(jnp.float32)
    er = lax.broadcasted_iota(jnp.int32, (E, E), 0)
    ec = lax.broadcasted_iota(jnp.int32, (E, E), 1)
    lt8 = (er < ec).astype(jnp.float32)
    off = jnp.dot(pcf, lt8, preferred_element_type=jnp.float32)

    BL = 128
    rr = lax.broadcasted_iota(jnp.int32, (BL, BL), 0)
    cc = lax.broadcasted_iota(jnp.int32, (BL, BL), 1)
    ltb = (cc < rr).astype(jnp.float32)
    run = jnp.zeros((1, E), jnp.float32)
    ranks = []
    for b in range(T // BL):
        sb = s[b * BL:(b + 1) * BL, :]
        ranks.append(jnp.dot(ltb, sb, preferred_element_type=jnp.float32) + run)
        run = run + jnp.sum(sb, axis=0, keepdims=True)
    rank = jnp.concatenate(ranks, axis=0)

    base = off + rank
    pos1 = jnp.sum(mask1 * base, axis=-1, keepdims=True)
    pos2 = jnp.sum(mask2 * (base + mask1), axis=-1, keepdims=True)
    pos_ref[...] = jnp.concatenate([pos1, pos2], axis=1).astype(jnp.int32)

    jm = lax.broadcasted_iota(jnp.int32, (1, NT), 1).astype(jnp.float32) * M
    ei = lax.broadcasted_iota(jnp.int32, (1, E), 1)
    texp = jnp.zeros((1, NT), jnp.float32)
    for e in range(E):
        sel = (ei == e).astype(jnp.float32)
        off_e = jnp.sum(off * sel, axis=-1, keepdims=True)
        pc_e = jnp.sum(pcf * sel, axis=-1, keepdims=True)
        texp += jnp.where((jm >= off_e) & (jm < off_e + pc_e), float(e), 0.0)
    total = jnp.sum(pcf, axis=-1, keepdims=True)
    last_e = jnp.max(jnp.where(pci > 0, ei, 0), axis=-1, keepdims=True)
    texp_ref[...] = jnp.where(jm < total, texp,
                              last_e.astype(jnp.float32)).astype(jnp.int32)

    avg = jnp.sum(p, axis=0, keepdims=True) / T
    mean = jnp.sum(avg) / E
    var = jnp.sum((avg - mean) ** 2) / (E - 1)
    loss_ref[...] = jnp.broadcast_to(LBW * var, (1, 1))


def _router(x2d, Wr, br2):
    return pl.pallas_call(
        _router_body,
        in_specs=[
            pl.BlockSpec((T, H), lambda: (0, 0)),
            pl.BlockSpec((E, H), lambda: (0, 0)),
            pl.BlockSpec((1, E), lambda: (0, 0)),
        ],
        out_specs=[
            pl.BlockSpec((T, K), lambda: (0, 0)),
            pl.BlockSpec((T, 16), lambda: (0, 0)),
            pl.BlockSpec((T, 16), lambda: (0, 0)),
            pl.BlockSpec((1, NT), lambda: (0, 0)),
            pl.BlockSpec((1, 1), lambda: (0, 0)),
        ],
        out_shape=[
            jax.ShapeDtypeStruct((T, K), jnp.int32),
            jax.ShapeDtypeStruct((T, 16), jnp.float32),
            jax.ShapeDtypeStruct((T, 16), jnp.float32),
            jax.ShapeDtypeStruct((1, NT), jnp.int32),
            jax.ShapeDtypeStruct((1, 1), jnp.float32),
        ],
    )(x2d, Wr, br2)


def _dispatch_body(x_hbm, pos_hbm, xg_hbm, idx0_v, idx1_v, rows_v, s0, s1):
    wid = lax.axis_index("s") * NC + lax.axis_index("c")
    tbase = wid * TPW
    pltpu.sync_copy(pos_hbm.at[0, pl.ds(tbase, TPW)], idx0_v)
    pltpu.sync_copy(pos_hbm.at[1, pl.ds(tbase, TPW)], idx1_v)
    pltpu.sync_copy(x_hbm.at[pl.ds(tbase, TPW), :], rows_v)
    c0 = pltpu.async_copy(rows_v, xg_hbm.at[idx0_v], s0)
    c1 = pltpu.async_copy(rows_v, xg_hbm.at[idx1_v], s1)
    c0.wait()
    c1.wait()


@functools.cache
def _make_dispatch():
    return pl.kernel(
        _dispatch_body,
        mesh=plsc.VectorSubcoreMesh(core_axis_name="c", subcore_axis_name="s"),
        out_type=jax.ShapeDtypeStruct((NTM, H), jnp.float32),
        scratch_types=[
            pltpu.VMEM((TPW,), jnp.int32),
            pltpu.VMEM((TPW,), jnp.int32),
            pltpu.VMEM((TPW, H), jnp.float32),
            pltpu.SemaphoreType.DMA,
            pltpu.SemaphoreType.DMA,
        ],
    )


def _gmm_body(texp_ref, xg_ref, w1_ref, b1_ref, w2_ref, b2_ref, out_ref):
    f = pl.program_id(1)
    h = jnp.dot(xg_ref[...], w1_ref[0].T, preferred_element_type=jnp.float32)
    h = h + b1_ref[0, 0]
    h = 0.5 * h * (1.0 + lax.erf(h * 0.7071067811865476))
    yc = jnp.dot(h, w2_ref[0].T, preferred_element_type=jnp.float32)

    @pl.when(f == 0)
    def _():
        out_ref[...] = yc + b2_ref[0]

    @pl.when(f != 0)
    def _():
        out_ref[...] += yc


def _gmm(texp, x_g, W1, b1, W2, b2):
    grid_spec = pltpu.PrefetchScalarGridSpec(
        num_scalar_prefetch=1,
        grid=(NT, NF),
        in_specs=[
            pl.BlockSpec((M, H), lambda j, f, tx: (j, 0)),
            pl.BlockSpec((1, FC, H), lambda j, f, tx: (tx[j], f, 0)),
            pl.BlockSpec((1, 1, 1, FC), lambda j, f, tx: (tx[j], f, 0, 0)),
            pl.BlockSpec((1, H, FC), lambda j, f, tx: (tx[j], 0, f)),
            pl.BlockSpec((1, 1, H), lambda j, f, tx: (tx[j], 0, 0)),
        ],
        out_specs=pl.BlockSpec((M, H), lambda j, f, tx: (j, 0)),
    )
    return pl.pallas_call(
        _gmm_body,
        grid_spec=grid_spec,
        out_shape=jax.ShapeDtypeStruct((NTM, H), jnp.float32),
    )(texp, x_g, W1, b1.reshape(E, NF, 1, FC), W2, b2.reshape(E, 1, H))


def _combine_body(yg_hbm, pos_hbm, w0b_hbm, w1b_hbm, out_hbm,
                  idx0_v, idx1_v, w0_v, w1_v, r0_v, r1_v, ob_v, s0, s1):
    wid = lax.axis_index("s") * NC + lax.axis_index("c")
    tbase = wid * TPW
    for half in range(NCH):
        base = tbase + half * CH
        pltpu.sync_copy(pos_hbm.at[0, pl.ds(base, CH)], idx0_v)
        pltpu.sync_copy(pos_hbm.at[1, pl.ds(base, CH)], idx1_v)
        pltpu.sync_copy(w0b_hbm.at[pl.ds(base, CH), :], w0_v)
        pltpu.sync_copy(w1b_hbm.at[pl.ds(base, CH), :], w1_v)
        c0 = pltpu.async_copy(yg_hbm.at[idx0_v], r0_v, s0)
        c1 = pltpu.async_copy(yg_hbm.at[idx1_v], r1_v, s1)
        c0.wait()
        c1.wait()
        for tok in range(CH):
            w0s = w0_v[tok, :]
            w1s = w1_v[tok, :]

            def body(j, _, tok=tok, w0s=w0s, w1s=w1s):
                a = r0_v[tok, pl.ds(j * 16, 16)]
                b = r1_v[tok, pl.ds(j * 16, 16)]
                ob_v[tok, pl.ds(j * 16, 16)] = a * w0s + b * w1s
                return 0

            lax.fori_loop(0, H // 16, body, 0)
        pltpu.sync_copy(ob_v, out_hbm.at[pl.ds(base, CH), :])


@functools.cache
def _make_combine():
    return pl.kernel(
        _combine_body,
        mesh=plsc.VectorSubcoreMesh(core_axis_name="c", subcore_axis_name="s"),
        out_type=jax.ShapeDtypeStruct((T, H), jnp.float32),
        scratch_types=[
            pltpu.VMEM((CH,), jnp.int32),
            pltpu.VMEM((CH,), jnp.int32),
            pltpu.VMEM((CH, 16), jnp.float32),
            pltpu.VMEM((CH, 16), jnp.float32),
            pltpu.VMEM((CH, H), jnp.float32),
            pltpu.VMEM((CH, H), jnp.float32),
            pltpu.VMEM((CH, H), jnp.float32),
            pltpu.SemaphoreType.DMA,
            pltpu.SemaphoreType.DMA,
        ],
    )


@jax.jit
def _moe(x2d, Wr, br2, W1, b1, W2, b2):
    pos, w0b, w1b, texp, loss = _router(x2d, Wr, br2)
    pos2t = pos.T
    x_g = _make_dispatch()(x2d, pos2t)
    y_g = _gmm(texp.reshape(NT), x_g, W1, b1, W2, b2)
    out = _make_combine()(y_g, pos2t, w0b, w1b)
    return out, loss


def kernel(x, Wr, br, W1, b1, W2, b2):
    Bq, Sq, Hq = x.shape
    x2d = x.reshape(-1, Hq)
    br2 = br.reshape(1, E)
    out, loss = _moe(x2d, Wr, br2, W1, b1, W2, b2)
    return out.reshape(Bq, Sq, Hq), loss.reshape(())
